# baseline (speedup 1.0000x reference)
import functools

import jax
import jax.numpy as jnp
from jax import lax
from jax.experimental import pallas as pl
from jax.experimental.pallas import tpu as pltpu
from jax.experimental.pallas import tpu_sc as plsc

_N, _E, _D = 10000, 320000, 128
_TILES = 32
_NPAD = 10240
_RPT = _NPAD // 16
_CH = 128
_NB = 2
_SG = 8
_NSG = 10
_NCH = _SG * _NSG
_EPT = _NCH * _CH
_EPAD = _EPT * _TILES
_BN = 1024


@functools.cache
def _sc_aggregate(with_deg):
    mesh = plsc.VectorSubcoreMesh(
        core_axis_name="c", subcore_axis_name="s", num_cores=2, num_subcores=16
    )

    def body(h_hbm, src_hbm, dst_hbm, zrow_hbm, zdeg_hbm, ones_hbm, *rest):
        if with_deg:
            (acc_out, deg_out, sidx, didx, rows_v, ones_v, acc_sh, deg_sh,
             gsem) = rest
        else:
            (acc_out, sidx, didx, rows_v, ones_v, acc_sh, gsem) = rest
        c = lax.axis_index("c")
        s = lax.axis_index("s")
        tid = c * 16 + s
        pltpu.sync_copy(ones_hbm, ones_v)
        pltpu.sync_copy(src_hbm.at[tid], sidx)
        pltpu.sync_copy(dst_hbm.at[tid], didx)
        pltpu.sync_copy(zrow_hbm, acc_sh.at[pl.ds(s * _RPT, _RPT)])
        if with_deg:
            pltpu.sync_copy(zdeg_hbm, deg_sh.at[pl.ds(s * _RPT, _RPT)])
        plsc.subcore_barrier()

        def chunk(i, carry):
            pltpu.async_copy(h_hbm.at[sidx.at[i]], rows_v, gsem).wait()
            pltpu.sync_copy(rows_v, acc_sh.at[didx.at[i]], add=True)
            if with_deg:
                pltpu.sync_copy(ones_v, deg_sh.at[didx.at[i]], add=True)
            return carry

        lax.fori_loop(0, _NCH, chunk, 0)
        plsc.subcore_barrier()
        pltpu.sync_copy(acc_sh.at[pl.ds(s * _RPT, _RPT)],
                        acc_out.at[c, pl.ds(s * _RPT, _RPT)])
        if with_deg:
            pltpu.sync_copy(deg_sh.at[pl.ds(s * _RPT, _RPT)],
                            deg_out.at[c, pl.ds(s * _RPT, _RPT)])

    out_type = [jax.ShapeDtypeStruct((2, _NPAD, _D), jnp.float32)]
    scratch = [
        pltpu.VMEM((_NCH, _CH), jnp.int32),
        pltpu.VMEM((_NCH, _CH), jnp.int32),
        pltpu.VMEM((_CH, _D), jnp.float32),
        pltpu.VMEM((_CH,), jnp.float32),
        pltpu.VMEM_SHARED((_NPAD, _D), jnp.float32),
    ]
    if with_deg:
        out_type.append(jax.ShapeDtypeStruct((2, _NPAD), jnp.float32))
        scratch.append(pltpu.VMEM_SHARED((_NPAD,), jnp.float32))
    scratch += [
        pltpu.SemaphoreType.DMA,
    ]
    return pl.kernel(body, out_type=tuple(out_type), mesh=mesh,
                     scratch_types=scratch)


def _layer_body(relu, h_ref, acc_ref, d0_ref, d1_ref, ws_ref, wn_ref, b_ref,
                o_ref):
    inv = 1.0 / jnp.maximum(d0_ref[...] + d1_ref[...], 1.0)
    hn = (acc_ref[0] + acc_ref[1]) * inv
    out = (jnp.dot(h_ref[...], ws_ref[...], preferred_element_type=jnp.float32)
           + jnp.dot(hn, wn_ref[...], preferred_element_type=jnp.float32)
           + b_ref[...])
    if relu:
        out = jnp.maximum(out, 0.0)
    o_ref[...] = out


def _layer_call(h, acc, d0, d1, ws, wn, b, relu):
    return pl.pallas_call(
        functools.partial(_layer_body, relu),
        grid=(_NPAD // _BN,),
        in_specs=[
            pl.BlockSpec((_BN, _D), lambda i: (i, 0)),
            pl.BlockSpec((2, _BN, _D), lambda i: (0, i, 0)),
            pl.BlockSpec((_BN, 1), lambda i: (i, 0)),
            pl.BlockSpec((_BN, 1), lambda i: (i, 0)),
            pl.BlockSpec((_D, _D), lambda i: (0, 0)),
            pl.BlockSpec((_D, _D), lambda i: (0, 0)),
            pl.BlockSpec((1, _D), lambda i: (0, 0)),
        ],
        out_specs=pl.BlockSpec((_BN, _D), lambda i: (i, 0)),
        out_shape=jax.ShapeDtypeStruct((_NPAD, _D), jnp.float32),
    )(h, acc, d0, d1, ws, wn, b)


def kernel(x, edge_index, W_self1, W_neigh1, b1, W_self2, W_neigh2, b2):
    src = edge_index[0]
    dst = edge_index[1]
    pad_e = _EPAD - _E
    src_p = jnp.concatenate(
        [src, jnp.zeros((pad_e,), jnp.int32)]).reshape(_TILES, _NCH, _CH)
    dst_p = jnp.concatenate(
        [dst, jnp.full((pad_e,), _N, jnp.int32)]).reshape(_TILES, _NCH, _CH)
    x_p = jnp.pad(x, ((0, _NPAD - _N), (0, 0)))
    zrow = jnp.zeros((_RPT, _D), jnp.float32)
    zdeg = jnp.zeros((_RPT,), jnp.float32)
    ones = jnp.ones((_CH,), jnp.float32)

    acc1, deg = _sc_aggregate(True)(x_p, src_p, dst_p, zrow, zdeg, ones)
    d0 = deg[0].reshape(_NPAD, 1)
    d1 = deg[1].reshape(_NPAD, 1)
    h1 = _layer_call(x_p, acc1, d0, d1, W_self1, W_neigh1,
                     b1.reshape(1, _D), relu=True)
    (acc2,) = _sc_aggregate(False)(h1, src_p, dst_p, zrow, zdeg, ones)
    h2 = _layer_call(h1, acc2, d0, d1, W_self2, W_neigh2,
                     b2.reshape(1, _D), relu=False)

    fl = (_N * (4 * _D * _D) + _E * 2 * _D) / 1e12
    total_flops = jnp.asarray(fl + fl, dtype=jnp.float32)
    return h2[:_N], total_flops


# speedup vs baseline: 1.0005x; 1.0005x over previous
import functools

import jax
import jax.numpy as jnp
from jax import lax
from jax.experimental import pallas as pl
from jax.experimental.pallas import tpu as pltpu
from jax.experimental.pallas import tpu_sc as plsc

_N, _E, _D = 10000, 320000, 128
_TILES = 32
_NPAD = 10240
_RPT = _NPAD // 16
_CH = 128
_NB = 2
_SG = 8
_NSG = 10
_NCH = _SG * _NSG
_EPT = _NCH * _CH
_EPAD = _EPT * _TILES
_BN = 1024


@functools.cache
def _sc_aggregate(with_deg):
    mesh = plsc.VectorSubcoreMesh(
        core_axis_name="c", subcore_axis_name="s", num_cores=2, num_subcores=16
    )

    def body(h_hbm, src_hbm, dst_hbm, zrow_hbm, zdeg_hbm, ones_hbm, *rest):
        if with_deg:
            (acc_out, deg_out, src_a, dst_a, src_b, dst_b, rows_v, ones_v,
             acc_sh, deg_sh, gsem, isem) = rest
        else:
            (acc_out, src_a, dst_a, src_b, dst_b, rows_v, ones_v,
             acc_sh, gsem, isem) = rest
        c = lax.axis_index("c")
        s = lax.axis_index("s")
        tid = c * 16 + s
        base = tid * _EPT
        pltpu.sync_copy(ones_hbm, ones_v)
        pltpu.sync_copy(src_hbm.at[pl.ds(base, _CH)], src_a)
        pltpu.sync_copy(dst_hbm.at[pl.ds(base, _CH)], dst_a)
        pltpu.sync_copy(zrow_hbm, acc_sh.at[pl.ds(s * _RPT, _RPT)])
        if with_deg:
            pltpu.sync_copy(zdeg_hbm, deg_sh.at[pl.ds(s * _RPT, _RPT)])
        plsc.subcore_barrier()

        def do_chunk(sv, dv):
            pltpu.async_copy(h_hbm.at[sv], rows_v, gsem).wait()
            pltpu.sync_copy(rows_v, acc_sh.at[dv], add=True)
            if with_deg:
                pltpu.sync_copy(ones_v, deg_sh.at[dv], add=True)

        def pair(i, carry):
            offb = base + (2 * i + 1) * _CH
            pbs = pltpu.async_copy(src_hbm.at[pl.ds(offb, _CH)], src_b,
                                   isem.at[0])
            pbd = pltpu.async_copy(dst_hbm.at[pl.ds(offb, _CH)], dst_b,
                                   isem.at[1])
            do_chunk(src_a, dst_a)
            pbs.wait()
            pbd.wait()
            offa = base + jnp.where(i + 1 < _NCH // 2, (2 * i + 2) * _CH, 0)
            pas = pltpu.async_copy(src_hbm.at[pl.ds(offa, _CH)], src_a,
                                   isem.at[0])
            pad = pltpu.async_copy(dst_hbm.at[pl.ds(offa, _CH)], dst_a,
                                   isem.at[1])
            do_chunk(src_b, dst_b)
            pas.wait()
            pad.wait()
            return carry

        lax.fori_loop(0, _NCH // 2, pair, 0)
        plsc.subcore_barrier()
        pltpu.sync_copy(acc_sh.at[pl.ds(s * _RPT, _RPT)],
                        acc_out.at[c, pl.ds(s * _RPT, _RPT)])
        if with_deg:
            pltpu.sync_copy(deg_sh.at[pl.ds(s * _RPT, _RPT)],
                            deg_out.at[c, pl.ds(s * _RPT, _RPT)])

    out_type = [jax.ShapeDtypeStruct((2, _NPAD, _D), jnp.float32)]
    scratch = [
        pltpu.VMEM((_CH,), jnp.int32),
        pltpu.VMEM((_CH,), jnp.int32),
        pltpu.VMEM((_CH,), jnp.int32),
        pltpu.VMEM((_CH,), jnp.int32),
        pltpu.VMEM((_CH, _D), jnp.float32),
        pltpu.VMEM((_CH,), jnp.float32),
        pltpu.VMEM_SHARED((_NPAD, _D), jnp.float32),
    ]
    if with_deg:
        out_type.append(jax.ShapeDtypeStruct((2, _NPAD), jnp.float32))
        scratch.append(pltpu.VMEM_SHARED((_NPAD,), jnp.float32))
    scratch += [
        pltpu.SemaphoreType.DMA,
        pltpu.SemaphoreType.DMA((2,)),
    ]
    return pl.kernel(body, out_type=tuple(out_type), mesh=mesh,
                     scratch_types=scratch)


def _layer_body(relu, h_ref, acc_ref, d0_ref, d1_ref, ws_ref, wn_ref, b_ref,
                o_ref):
    inv = 1.0 / jnp.maximum(d0_ref[...] + d1_ref[...], 1.0)
    hn = (acc_ref[0] + acc_ref[1]) * inv
    out = (jnp.dot(h_ref[...], ws_ref[...], preferred_element_type=jnp.float32)
           + jnp.dot(hn, wn_ref[...], preferred_element_type=jnp.float32)
           + b_ref[...])
    if relu:
        out = jnp.maximum(out, 0.0)
    o_ref[...] = out


def _layer_call(h, acc, d0, d1, ws, wn, b, relu):
    return pl.pallas_call(
        functools.partial(_layer_body, relu),
        grid=(_NPAD // _BN,),
        in_specs=[
            pl.BlockSpec((_BN, _D), lambda i: (i, 0)),
            pl.BlockSpec((2, _BN, _D), lambda i: (0, i, 0)),
            pl.BlockSpec((_BN, 1), lambda i: (i, 0)),
            pl.BlockSpec((_BN, 1), lambda i: (i, 0)),
            pl.BlockSpec((_D, _D), lambda i: (0, 0)),
            pl.BlockSpec((_D, _D), lambda i: (0, 0)),
            pl.BlockSpec((1, _D), lambda i: (0, 0)),
        ],
        out_specs=pl.BlockSpec((_BN, _D), lambda i: (i, 0)),
        out_shape=jax.ShapeDtypeStruct((_NPAD, _D), jnp.float32),
    )(h, acc, d0, d1, ws, wn, b)


def kernel(x, edge_index, W_self1, W_neigh1, b1, W_self2, W_neigh2, b2):
    src = edge_index[0]
    dst = edge_index[1]
    pad_e = _EPAD - _E
    src_p = jnp.concatenate([src, jnp.zeros((pad_e,), jnp.int32)])
    dst_p = jnp.concatenate([dst, jnp.full((pad_e,), _N, jnp.int32)])
    x_p = jnp.pad(x, ((0, _NPAD - _N), (0, 0)))
    zrow = jnp.zeros((_RPT, _D), jnp.float32)
    zdeg = jnp.zeros((_RPT,), jnp.float32)
    ones = jnp.ones((_CH,), jnp.float32)

    acc1, deg = _sc_aggregate(True)(x_p, src_p, dst_p, zrow, zdeg, ones)
    d0 = deg[0].reshape(_NPAD, 1)
    d1 = deg[1].reshape(_NPAD, 1)
    h1 = _layer_call(x_p, acc1, d0, d1, W_self1, W_neigh1,
                     b1.reshape(1, _D), relu=True)
    (acc2,) = _sc_aggregate(False)(h1, src_p, dst_p, zrow, zdeg, ones)
    h2 = _layer_call(h1, acc2, d0, d1, W_self2, W_neigh2,
                     b2.reshape(1, _D), relu=False)

    fl = (_N * (4 * _D * _D) + _E * 2 * _D) / 1e12
    total_flops = jnp.asarray(fl + fl, dtype=jnp.float32)
    return h2[:_N], total_flops


# speedup vs baseline: 1.3585x; 1.3578x over previous
import functools

import jax
import jax.numpy as jnp
from jax import lax
from jax.experimental import pallas as pl
from jax.experimental.pallas import tpu as pltpu
from jax.experimental.pallas import tpu_sc as plsc

_N, _E, _D = 10000, 320000, 128
_TILES = 32
_NPAD = 10240
_RPT = _NPAD // 16
_CH = 128
_EPAD = -(-_E // (_TILES * _CH)) * _TILES * _CH
_EPT = _EPAD // _TILES
_NCH = _EPT // _CH
_BN = 1024


@functools.cache
def _sc_aggregate():
    mesh = plsc.VectorSubcoreMesh(
        core_axis_name="c", subcore_axis_name="s", num_cores=2, num_subcores=16
    )

    def body(h_hbm, src_hbm, dst_hbm, zrow_hbm, zdeg_hbm, ones_hbm,
             acc_out, deg_out,
             src_v, dst_v, rows_v, ones_v, acc_sh, deg_sh, sem):
        c = lax.axis_index("c")
        s = lax.axis_index("s")
        pltpu.sync_copy(ones_hbm, ones_v)
        pltpu.sync_copy(zrow_hbm, acc_sh.at[pl.ds(s * _RPT, _RPT)])
        pltpu.sync_copy(zdeg_hbm, deg_sh.at[pl.ds(s * _RPT, _RPT)])
        plsc.subcore_barrier()

        base = (c * 16 + s) * _EPT

        def chunk(i, carry):
            off = base + i * _CH
            pltpu.sync_copy(src_hbm.at[pl.ds(off, _CH)], src_v)
            pltpu.sync_copy(dst_hbm.at[pl.ds(off, _CH)], dst_v)
            pltpu.async_copy(h_hbm.at[src_v], rows_v, sem).wait()
            pltpu.sync_copy(rows_v, acc_sh.at[dst_v], add=True)
            pltpu.sync_copy(ones_v, deg_sh.at[dst_v], add=True)
            return carry

        lax.fori_loop(0, _NCH, chunk, 0)
        plsc.subcore_barrier()
        pltpu.sync_copy(acc_sh.at[pl.ds(s * _RPT, _RPT)],
                        acc_out.at[c, pl.ds(s * _RPT, _RPT)])
        pltpu.sync_copy(deg_sh.at[pl.ds(s * _RPT, _RPT)],
                        deg_out.at[c, pl.ds(s * _RPT, _RPT)])

    return pl.kernel(
        body,
        out_type=(jax.ShapeDtypeStruct((2, _NPAD, _D), jnp.float32),
                  jax.ShapeDtypeStruct((2, _NPAD), jnp.float32)),
        mesh=mesh,
        scratch_types=[
            pltpu.VMEM((_CH,), jnp.int32),
            pltpu.VMEM((_CH,), jnp.int32),
            pltpu.VMEM((_CH, _D), jnp.float32),
            pltpu.VMEM((_CH,), jnp.float32),
            pltpu.VMEM_SHARED((_NPAD, _D), jnp.float32),
            pltpu.VMEM_SHARED((_NPAD,), jnp.float32),
            pltpu.SemaphoreType.DMA,
        ],
    )


def _layer_body(relu, h_ref, acc_ref, d0_ref, d1_ref, ws_ref, wn_ref, b_ref,
                o_ref):
    inv = 1.0 / jnp.maximum(d0_ref[...] + d1_ref[...], 1.0)
    hn = (acc_ref[0] + acc_ref[1]) * inv
    out = (jnp.dot(h_ref[...], ws_ref[...], preferred_element_type=jnp.float32)
           + jnp.dot(hn, wn_ref[...], preferred_element_type=jnp.float32)
           + b_ref[...])
    if relu:
        out = jnp.maximum(out, 0.0)
    o_ref[...] = out


def _layer_call(h, acc, d0, d1, ws, wn, b, relu):
    return pl.pallas_call(
        functools.partial(_layer_body, relu),
        grid=(_NPAD // _BN,),
        in_specs=[
            pl.BlockSpec((_BN, _D), lambda i: (i, 0)),
            pl.BlockSpec((2, _BN, _D), lambda i: (0, i, 0)),
            pl.BlockSpec((_BN, 1), lambda i: (i, 0)),
            pl.BlockSpec((_BN, 1), lambda i: (i, 0)),
            pl.BlockSpec((_D, _D), lambda i: (0, 0)),
            pl.BlockSpec((_D, _D), lambda i: (0, 0)),
            pl.BlockSpec((1, _D), lambda i: (0, 0)),
        ],
        out_specs=pl.BlockSpec((_BN, _D), lambda i: (i, 0)),
        out_shape=jax.ShapeDtypeStruct((_NPAD, _D), jnp.float32),
    )(h, acc, d0, d1, ws, wn, b)


def kernel(x, edge_index, W_self1, W_neigh1, b1, W_self2, W_neigh2, b2):
    src = edge_index[0]
    dst = edge_index[1]
    pad_e = _EPAD - _E
    src_p = jnp.concatenate([src, jnp.zeros((pad_e,), jnp.int32)])
    dst_p = jnp.concatenate([dst, jnp.full((pad_e,), _N, jnp.int32)])
    x_p = jnp.pad(x, ((0, _NPAD - _N), (0, 0)))
    zrow = jnp.zeros((_RPT, _D), jnp.float32)
    zdeg = jnp.zeros((_RPT,), jnp.float32)
    ones = jnp.ones((_CH,), jnp.float32)

    agg = _sc_aggregate()
    acc1, deg = agg(x_p, src_p, dst_p, zrow, zdeg, ones)
    d0 = deg[0].reshape(_NPAD, 1)
    d1 = deg[1].reshape(_NPAD, 1)
    h1 = _layer_call(x_p, acc1, d0, d1, W_self1, W_neigh1,
                     b1.reshape(1, _D), relu=True)
    acc2, _ = agg(h1, src_p, dst_p, zrow, zdeg, ones)
    h2 = _layer_call(h1, acc2, d0, d1, W_self2, W_neigh2,
                     b2.reshape(1, _D), relu=False)

    fl = (_N * (4 * _D * _D) + _E * 2 * _D) / 1e12
    total_flops = jnp.asarray(fl + fl, dtype=jnp.float32)
    return h2[:_N], total_flops


# speedup vs baseline: 1.4251x; 1.0490x over previous
import functools

import jax
import jax.numpy as jnp
from jax import lax
from jax.experimental import pallas as pl
from jax.experimental.pallas import tpu as pltpu
from jax.experimental.pallas import tpu_sc as plsc

_N, _E, _D = 10000, 320000, 128
_TILES = 32
_NPAD = 10240
_RPT = _NPAD // 16
_CH = 128
_EPAD = -(-_E // (_TILES * _CH)) * _TILES * _CH
_EPT = _EPAD // _TILES
_NCH = _EPT // _CH
_BN = 1024


@functools.cache
def _sc_aggregate(with_deg):
    mesh = plsc.VectorSubcoreMesh(
        core_axis_name="c", subcore_axis_name="s", num_cores=2, num_subcores=16
    )

    def body(h_hbm, src_hbm, dst_hbm, zrow_hbm, zdeg_hbm, ones_hbm, *rest):
        if with_deg:
            (acc_out, deg_out,
             src_v, dst_v, rows_v, ones_v, acc_sh, deg_sh, sem) = rest
        else:
            (acc_out, src_v, dst_v, rows_v, ones_v, acc_sh, sem) = rest
        c = lax.axis_index("c")
        s = lax.axis_index("s")
        tid = c * 16 + s
        pltpu.sync_copy(ones_hbm, ones_v)
        pltpu.sync_copy(zrow_hbm, acc_sh.at[pl.ds(s * _RPT, _RPT)])
        if with_deg:
            pltpu.sync_copy(zdeg_hbm, deg_sh.at[pl.ds(s * _RPT, _RPT)])
        plsc.subcore_barrier()

        base = tid * _EPT

        def chunk(i, carry):
            off = base + i * _CH
            pltpu.sync_copy(src_hbm.at[pl.ds(off, _CH)], src_v)
            pltpu.sync_copy(dst_hbm.at[pl.ds(off, _CH)], dst_v)
            pltpu.async_copy(h_hbm.at[src_v], rows_v, sem).wait()
            pltpu.sync_copy(rows_v, acc_sh.at[dst_v], add=True)
            if with_deg:
                pltpu.sync_copy(ones_v, deg_sh.at[dst_v], add=True)
            return carry

        lax.fori_loop(0, _NCH, chunk, 0)
        plsc.subcore_barrier()
        pltpu.sync_copy(acc_sh.at[pl.ds(s * _RPT, _RPT)],
                        acc_out.at[c, pl.ds(s * _RPT, _RPT)])
        if with_deg:
            pltpu.sync_copy(deg_sh.at[pl.ds(s * _RPT, _RPT)],
                            deg_out.at[c, pl.ds(s * _RPT, _RPT)])

    out_type = [jax.ShapeDtypeStruct((2, _NPAD, _D), jnp.float32)]
    scratch = [
        pltpu.VMEM((_CH,), jnp.int32),
        pltpu.VMEM((_CH,), jnp.int32),
        pltpu.VMEM((_CH, _D), jnp.float32),
        pltpu.VMEM((_CH,), jnp.float32),
        pltpu.VMEM_SHARED((_NPAD, _D), jnp.float32),
    ]
    if with_deg:
        out_type.append(jax.ShapeDtypeStruct((2, _NPAD), jnp.float32))
        scratch.append(pltpu.VMEM_SHARED((_NPAD,), jnp.float32))
    scratch.append(pltpu.SemaphoreType.DMA)
    return pl.kernel(body, out_type=tuple(out_type), mesh=mesh,
                     scratch_types=scratch)


def _layer_body(relu, h_ref, acc_ref, d0_ref, d1_ref, ws_ref, wn_ref, b_ref,
                o_ref):
    inv = 1.0 / jnp.maximum(d0_ref[...] + d1_ref[...], 1.0)
    hn = (acc_ref[0] + acc_ref[1]) * inv
    out = (jnp.dot(h_ref[...], ws_ref[...], preferred_element_type=jnp.float32)
           + jnp.dot(hn, wn_ref[...], preferred_element_type=jnp.float32)
           + b_ref[...])
    if relu:
        out = jnp.maximum(out, 0.0)
    o_ref[...] = out


def _layer_call(h, acc, d0, d1, ws, wn, b, relu):
    return pl.pallas_call(
        functools.partial(_layer_body, relu),
        grid=(_NPAD // _BN,),
        in_specs=[
            pl.BlockSpec((_BN, _D), lambda i: (i, 0)),
            pl.BlockSpec((2, _BN, _D), lambda i: (0, i, 0)),
            pl.BlockSpec((_BN, 1), lambda i: (i, 0)),
            pl.BlockSpec((_BN, 1), lambda i: (i, 0)),
            pl.BlockSpec((_D, _D), lambda i: (0, 0)),
            pl.BlockSpec((_D, _D), lambda i: (0, 0)),
            pl.BlockSpec((1, _D), lambda i: (0, 0)),
        ],
        out_specs=pl.BlockSpec((_BN, _D), lambda i: (i, 0)),
        out_shape=jax.ShapeDtypeStruct((_NPAD, _D), jnp.float32),
    )(h, acc, d0, d1, ws, wn, b)


def kernel(x, edge_index, W_self1, W_neigh1, b1, W_self2, W_neigh2, b2):
    src = edge_index[0]
    dst = edge_index[1]
    pad_e = _EPAD - _E
    src_p = jnp.concatenate([src, jnp.zeros((pad_e,), jnp.int32)])
    dst_p = jnp.concatenate([dst, jnp.full((pad_e,), _N, jnp.int32)])
    x_p = jnp.pad(x, ((0, _NPAD - _N), (0, 0)))
    zrow = jnp.zeros((_RPT, _D), jnp.float32)
    zdeg = jnp.zeros((_RPT,), jnp.float32)
    ones = jnp.ones((_CH,), jnp.float32)

    acc1, deg = _sc_aggregate(True)(x_p, src_p, dst_p, zrow, zdeg, ones)
    d0 = deg[0].reshape(_NPAD, 1)
    d1 = deg[1].reshape(_NPAD, 1)
    h1 = _layer_call(x_p, acc1, d0, d1, W_self1, W_neigh1,
                     b1.reshape(1, _D), relu=True)
    (acc2,) = _sc_aggregate(False)(h1, src_p, dst_p, zrow, zdeg, ones)
    h2 = _layer_call(h1, acc2, d0, d1, W_self2, W_neigh2,
                     b2.reshape(1, _D), relu=False)

    fl = (_N * (4 * _D * _D) + _E * 2 * _D) / 1e12
    total_flops = jnp.asarray(fl + fl, dtype=jnp.float32)
    return h2[:_N], total_flops
